# baseline (speedup 1.0000x reference)
import functools

import jax
import jax.numpy as jnp
from jax import lax
from jax.experimental import pallas as pl
from jax.experimental.pallas import tpu as pltpu
from jax.experimental.pallas import tpu_sc as plsc

HIDDEN = 64
BATCH = 4096
NUM_CORES = 2
NUM_SUBCORES = 16
LANES = 16
NUM_WORKERS = NUM_CORES * NUM_SUBCORES
BPW = BATCH // NUM_WORKERS
HALF = BPW // 2
HGROUPS = HALF // LANES
WIN = 8


@functools.partial(
    pl.kernel,
    mesh=plsc.VectorSubcoreMesh(core_axis_name="c", subcore_axis_name="s"),
    out_type=jax.ShapeDtypeStruct((BATCH,), jnp.float32),
    compiler_params=pltpu.CompilerParams(
        needs_layout_passes=False, use_tc_tiling_on_sc=False),
    scratch_types=[
        pltpu.VMEM((BPW,), jnp.int32),
        pltpu.VMEM((BPW,), jnp.int32),
        pltpu.VMEM((HIDDEN, HALF * WIN), jnp.float32),
        pltpu.VMEM((HIDDEN, HALF * WIN), jnp.float32),
        pltpu.VMEM((BPW,), jnp.float32),
        pltpu.SemaphoreType.DMA,
        pltpu.SemaphoreType.DMA,
    ],
)
def _mf_sc(uidx_hbm, iidx_hbm, utT_hbm, itT_hbm, out_hbm,
           uidx_v, iidx_v, ubuf_v, ibuf_v, res_v, usem, isem):
    wid = lax.axis_index("s") * NUM_CORES + lax.axis_index("c")
    base = wid * BPW

    pltpu.sync_copy(uidx_hbm.at[pl.ds(base, BPW)], uidx_v)
    pltpu.sync_copy(iidx_hbm.at[pl.ds(base, BPW)], iidx_v)

    iota16 = lax.broadcasted_iota(jnp.int32, (LANES,), 0)

    for h in range(2):
        for g in range(HGROUPS):
            uvec = uidx_v[pl.ds(h * HALF + g * LANES, LANES)]
            ivec = iidx_v[pl.ds(h * HALF + g * LANES, LANES)]
            ucps = []
            icps = []
            for l in range(LANES):
                slot = g * LANES + l
                ur8 = pl.multiple_of((uvec[l] // WIN) * WIN, WIN)
                ir8 = pl.multiple_of((ivec[l] // WIN) * WIN, WIN)
                ucps.append(pltpu.async_copy(
                    utT_hbm.at[:, pl.ds(ur8, WIN)],
                    ubuf_v.at[:, pl.ds(slot * WIN, WIN)], usem))
                icps.append(pltpu.async_copy(
                    itT_hbm.at[:, pl.ds(ir8, WIN)],
                    ibuf_v.at[:, pl.ds(slot * WIN, WIN)], isem))
            for cp in ucps:
                cp.wait()
            for cp in icps:
                cp.wait()

        for g in range(HGROUPS):
            uvec = uidx_v[pl.ds(h * HALF + g * LANES, LANES)]
            ivec = iidx_v[pl.ds(h * HALF + g * LANES, LANES)]
            slot_base = (g * LANES) * WIN + iota16 * WIN
            ucol = slot_base + (uvec - (uvec // WIN) * WIN)
            icol = slot_base + (ivec - (ivec // WIN) * WIN)

            def feat_body(c, acc, _ucol=ucol, _icol=icol):
                crow = jnp.full((LANES,), 0, jnp.int32) + c
                uc = plsc.load_gather(ubuf_v, [crow, _ucol])
                ic = plsc.load_gather(ibuf_v, [crow, _icol])
                return acc + uc * ic

            acc0 = jnp.zeros((LANES,), jnp.float32)
            tot = lax.fori_loop(0, HIDDEN, feat_body, acc0)
            res_v[pl.ds(h * HALF + g * LANES, LANES)] = (
                1.0 / (1.0 + jnp.exp(-tot)))

    pltpu.sync_copy(res_v, out_hbm.at[pl.ds(base, BPW)])


def kernel(x, user_table, item_table):
    uidx = x[:, 0].astype(jnp.int32)
    iidx = x[:, 1].astype(jnp.int32)
    return _mf_sc(uidx, iidx, user_table.T, item_table.T)


# speedup vs baseline: 3.1382x; 3.1382x over previous
import functools

import jax
import jax.numpy as jnp
from jax import lax
from jax.experimental import pallas as pl
from jax.experimental.pallas import tpu as pltpu
from jax.experimental.pallas import tpu_sc as plsc

HIDDEN = 64
WORDS = HIDDEN // 2
BATCH = 4096
NUM_CORES = 2
NUM_SUBCORES = 16
LANES = 16
NUM_WORKERS = NUM_CORES * NUM_SUBCORES
BPW = BATCH // NUM_WORKERS
GROUPS = BPW // LANES
ACC_PITCH = LANES + 1


@functools.partial(
    pl.kernel,
    mesh=plsc.VectorSubcoreMesh(core_axis_name="c", subcore_axis_name="s"),
    out_type=jax.ShapeDtypeStruct((BATCH,), jnp.float32),
    compiler_params=pltpu.CompilerParams(
        needs_layout_passes=False, use_tc_tiling_on_sc=False),
    scratch_types=[
        pltpu.VMEM((BPW,), jnp.int32),
        pltpu.VMEM((BPW,), jnp.int32),
        pltpu.VMEM((BPW, WORDS), jnp.float32),
        pltpu.VMEM((BPW, WORDS), jnp.float32),
        pltpu.VMEM((BPW, ACC_PITCH), jnp.float32),
        pltpu.VMEM((BPW,), jnp.float32),
        pltpu.SemaphoreType.DMA,
        pltpu.SemaphoreType.DMA,
    ],
)
def _mf_sc(uidx_hbm, iidx_hbm, ut_hbm, it_hbm, out_hbm,
           uidx_v, iidx_v, ubuf_v, ibuf_v, acc_v, res_v, usem, isem):
    wid = lax.axis_index("s") * NUM_CORES + lax.axis_index("c")
    base = wid * BPW

    pltpu.sync_copy(uidx_hbm.at[pl.ds(base, BPW)], uidx_v)
    pltpu.sync_copy(iidx_hbm.at[pl.ds(base, BPW)], iidx_v)

    ucp = pltpu.async_copy(ut_hbm.at[uidx_v], ubuf_v, usem)
    icp = pltpu.async_copy(it_hbm.at[iidx_v], ibuf_v, isem)
    ucp.wait()
    icp.wait()

    def row_body(r, _):
        acc = None
        for j in range(WORDS // LANES):
            uw = ubuf_v[r, pl.ds(j * LANES, LANES)]
            iw = ibuf_v[r, pl.ds(j * LANES, LANES)]
            ua, ub = plsc.unpack(plsc.bitcast(uw, jnp.bfloat16),
                                 format=plsc.PackFormat.INTERLEAVED)
            ia, ib = plsc.unpack(plsc.bitcast(iw, jnp.bfloat16),
                                 format=plsc.PackFormat.INTERLEAVED)
            term = ua * ia + ub * ib
            acc = term if acc is None else acc + term
        acc_v[r, pl.ds(0, LANES)] = acc
        return _

    lax.fori_loop(0, BPW, row_body, None)

    iota16 = lax.broadcasted_iota(jnp.int32, (LANES,), 0)
    for g in range(GROUPS):
        rows = iota16 + (g * LANES)
        tot = plsc.load_gather(acc_v, [rows, jnp.zeros((LANES,), jnp.int32)])
        for j in range(1, LANES):
            tot = tot + plsc.load_gather(
                acc_v, [rows, jnp.full((LANES,), j, jnp.int32)])
        res_v[pl.ds(g * LANES, LANES)] = 1.0 / (1.0 + jnp.exp(-tot))

    pltpu.sync_copy(res_v, out_hbm.at[pl.ds(base, BPW)])


def kernel(x, user_table, item_table):
    uidx = x[:, 0].astype(jnp.int32)
    iidx = x[:, 1].astype(jnp.int32)
    ut = lax.bitcast_convert_type(
        user_table.astype(jnp.bfloat16).reshape(-1, WORDS, 2), jnp.float32)
    it = lax.bitcast_convert_type(
        item_table.astype(jnp.bfloat16).reshape(-1, WORDS, 2), jnp.float32)
    return _mf_sc(uidx, iidx, ut, it)
